# Initial kernel scaffold; baseline (speedup 1.0000x reference)
import jax
import jax.numpy as jnp
from jax.experimental import pallas as pl


def kernel(x, edge_index, edge_weight, W1, b1, W2, b2):
    raise NotImplementedError("write your pallas kernel here")


# speedup vs baseline: 24.4305x; 24.4305x over previous
import functools

import jax
import jax.numpy as jnp
from jax import lax
from jax.experimental import pallas as pl
from jax.experimental.pallas import tpu as pltpu
from jax.experimental.pallas import tpu_sc as plsc

N = 10000
NPAD = 10240
IN_CH = 128
HID = 32
OUT = 16
E_RAW = 320000

NC, NS, L = 2, 16, 16
NW = NC * NS
SLICE = NPAD // NS
CHUNK = 128
CPT = 82
EPT = CPT * CHUNK
E_PAD = EPT * NW


def _mesh():
    return plsc.VectorSubcoreMesh(core_axis_name="c", subcore_axis_name="s",
                                  num_cores=NC, num_subcores=NS)


def _deg_call(dst3, ew3):
    @functools.partial(
        pl.kernel,
        out_type=jax.ShapeDtypeStruct((NC, NPAD), jnp.float32),
        mesh=_mesh(),
        scratch_types=[
            pltpu.VMEM((CPT, CHUNK), jnp.int32),
            pltpu.VMEM((CPT, CHUNK), jnp.float32),
            pltpu.VMEM((SLICE,), jnp.float32),
            pltpu.VMEM_SHARED((NPAD,), jnp.float32),
        ],
    )
    def deg_kernel(dst_hbm, ew_hbm, out_hbm, dst_v, ew_v, zbuf, acc):
        cid = lax.axis_index("c")
        sid = lax.axis_index("s")
        wid = sid * NC + cid

        def zb(i, c):
            zbuf[pl.ds(i * L, L)] = jnp.zeros((L,), jnp.float32)
            return c

        lax.fori_loop(0, SLICE // L, zb, 0)
        pltpu.sync_copy(zbuf, acc.at[pl.ds(sid * SLICE, SLICE)])
        pltpu.sync_copy(dst_hbm.at[wid], dst_v)
        pltpu.sync_copy(ew_hbm.at[wid], ew_v)
        plsc.subcore_barrier()

        def body(j, c):
            pltpu.sync_copy(ew_v.at[j], acc.at[dst_v.at[j]], add=True)
            return c

        lax.fori_loop(0, CPT, body, 0)
        plsc.subcore_barrier()
        pltpu.sync_copy(acc.at[pl.ds(sid * SLICE, SLICE)],
                        out_hbm.at[cid, pl.ds(sid * SLICE, SLICE)])

    return deg_kernel(dst3, ew3)


def _agg_call(src3, dst3, ew3, h, zrs, D):
    @functools.partial(
        pl.kernel,
        out_type=jax.ShapeDtypeStruct((NC, NPAD, D), jnp.float32),
        mesh=_mesh(),
        compiler_params=pltpu.CompilerParams(use_tc_tiling_on_sc=False),
        scratch_types=[
            pltpu.VMEM((CPT, CHUNK), jnp.int32),
            pltpu.VMEM((CPT, CHUNK), jnp.int32),
            pltpu.VMEM((CPT, CHUNK), jnp.float32),
            pltpu.VMEM((2, CHUNK, D), jnp.float32),
            pltpu.VMEM((CHUNK, D), jnp.float32),
            pltpu.VMEM_SHARED((NPAD, D), jnp.float32),
            pltpu.SemaphoreType.DMA,
            pltpu.SemaphoreType.DMA,
        ],
    )
    def agg_kernel(src_hbm, dst_hbm, ew_hbm, h_hbm, z_hbm, out_hbm,
                   src_v, dst_v, ew_v, rows, obuf, acc, sem0, sem1):
        cid = lax.axis_index("c")
        sid = lax.axis_index("s")
        wid = sid * NC + cid
        sems = (sem0, sem1)
        nsl = pl.ds(sid * SLICE, SLICE)
        pltpu.sync_copy(z_hbm.at[nsl], acc.at[nsl])
        pltpu.sync_copy(src_hbm.at[wid], src_v)
        pltpu.sync_copy(dst_hbm.at[wid], dst_v)
        pltpu.sync_copy(ew_hbm.at[wid], ew_v)
        plsc.subcore_barrier()

        def fire(j, b):
            pltpu.async_copy(h_hbm.at[src_v.at[j]], rows.at[b], sems[b])

        def wait(j, b):
            pltpu.make_async_copy(h_hbm.at[src_v.at[j]], rows.at[b],
                                  sems[b]).wait()

        def scale(j, b):
            def kb(g, c):
                ewv = ew_v[j, pl.ds(g * L, L)]
                for e in range(L):
                    k = g * L + e
                    s = ewv[e]
                    for f in range(D // L):
                        sl = pl.ds(f * L, L)
                        rows[b, k, sl] = rows[b, k, sl] * s
                return c

            lax.fori_loop(0, CHUNK // L, kb, 0)

        def scat(j, b):
            pltpu.sync_copy(rows.at[b], acc.at[dst_v.at[j]], add=True)

        def outer(j, c):
            fire(j, 0)
            wait(j, 0)
            scale(j, 0)
            scat(j, 0)
            return c

        lax.fori_loop(0, CPT, outer, 0)
        plsc.subcore_barrier()

        def oc(t, c):
            osl = pl.ds(sid * SLICE + t * CHUNK, CHUNK)
            pltpu.sync_copy(acc.at[osl], obuf)
            pltpu.sync_copy(obuf, out_hbm.at[cid, osl])
            return c

        lax.fori_loop(0, SLICE // CHUNK, oc, 0)

    return agg_kernel(src3, dst3, ew3, h, zrs)


def _tc_prep(degp, x_pad, W1):
    def body(deg_ref, x_ref, w_ref, dinv_ref, h_ref):
        deg = deg_ref[0, :] + deg_ref[1, :]
        dinv = jnp.where(deg > 0, lax.rsqrt(jnp.maximum(deg, 1e-12)), 0.0)
        dinv_ref[...] = dinv
        h = jnp.dot(x_ref[...], w_ref[...], preferred_element_type=jnp.float32)
        h_ref[...] = h * dinv[:, None]

    return pl.pallas_call(
        body,
        out_shape=(jax.ShapeDtypeStruct((NPAD,), jnp.float32),
                   jax.ShapeDtypeStruct((NPAD, HID), jnp.float32)),
    )(degp, x_pad, W1)


def _tc_mid(p1, dinv, b1, W2):
    def body(p_ref, dinv_ref, b_ref, w_ref, h2_ref):
        dinv = dinv_ref[...]
        s = (p_ref[0] + p_ref[1]) * dinv[:, None] + b_ref[...][None, :]
        a = jnp.maximum(s, 0.0)
        h2 = jnp.dot(a, w_ref[...], preferred_element_type=jnp.float32)
        h2_ref[...] = h2 * dinv[:, None]

    return pl.pallas_call(
        body,
        out_shape=jax.ShapeDtypeStruct((NPAD, OUT), jnp.float32),
    )(p1, dinv, b1, W2)


def _tc_fin(p2, dinv, b2):
    def body(p_ref, dinv_ref, b_ref, o_ref):
        o_ref[...] = ((p_ref[0] + p_ref[1]) * dinv_ref[...][:, None]
                      + b_ref[...][None, :])

    return pl.pallas_call(
        body,
        out_shape=jax.ShapeDtypeStruct((NPAD, OUT), jnp.float32),
    )(p2, dinv, b2)


def kernel(x, edge_index, edge_weight, W1, b1, W2, b2):
    loop = jnp.arange(N, dtype=jnp.int32)
    src = jnp.concatenate([edge_index[0].astype(jnp.int32), loop])
    dst = jnp.concatenate([edge_index[1].astype(jnp.int32), loop])
    ew = jnp.concatenate([edge_weight.astype(jnp.float32),
                          jnp.ones((N,), jnp.float32)])
    pad = E_PAD - (E_RAW + N)
    src3 = jnp.pad(src, (0, pad)).reshape(NW, CPT, CHUNK)
    dst3 = jnp.pad(dst, (0, pad)).reshape(NW, CPT, CHUNK)
    ew3 = jnp.pad(ew, (0, pad)).reshape(NW, CPT, CHUNK)
    x_pad = jnp.pad(x, ((0, NPAD - N), (0, 0)))

    degp = _deg_call(dst3, ew3)
    dinv, h1 = _tc_prep(degp, x_pad, W1)
    p1 = _agg_call(src3, dst3, ew3, h1,
                   jnp.zeros((NPAD, HID), jnp.float32), HID)
    h2 = _tc_mid(p1, dinv, b1, W2)
    p2 = _agg_call(src3, dst3, ew3, h2,
                   jnp.zeros((NPAD, OUT), jnp.float32), OUT)
    out = _tc_fin(p2, dinv, b2)
    return out[:N]


# speedup vs baseline: 32.9671x; 1.3494x over previous
import functools

import jax
import jax.numpy as jnp
from jax import lax
from jax.experimental import pallas as pl
from jax.experimental.pallas import tpu as pltpu
from jax.experimental.pallas import tpu_sc as plsc

N = 10000
NPAD = 10240
IN_CH = 128
HID = 32
OUT = 16
E_RAW = 320000

NC, NS, L = 2, 16, 16
NW = NC * NS
SLICE = NPAD // NS
CHUNK = 128
CPT = 82
EPT = CPT * CHUNK
E_PAD = EPT * NW


def _mesh():
    return plsc.VectorSubcoreMesh(core_axis_name="c", subcore_axis_name="s",
                                  num_cores=NC, num_subcores=NS)


def _deg_call(dst3, ew3):
    @functools.partial(
        pl.kernel,
        out_type=jax.ShapeDtypeStruct((NC, NPAD), jnp.float32),
        mesh=_mesh(),
        scratch_types=[
            pltpu.VMEM((CPT, CHUNK), jnp.int32),
            pltpu.VMEM((CPT, CHUNK), jnp.float32),
            pltpu.VMEM((SLICE,), jnp.float32),
            pltpu.VMEM_SHARED((NPAD,), jnp.float32),
        ],
    )
    def deg_kernel(dst_hbm, ew_hbm, out_hbm, dst_v, ew_v, zbuf, acc):
        cid = lax.axis_index("c")
        sid = lax.axis_index("s")
        wid = sid * NC + cid

        def zb(i, c):
            zbuf[pl.ds(i * L, L)] = jnp.zeros((L,), jnp.float32)
            return c

        lax.fori_loop(0, SLICE // L, zb, 0)
        pltpu.sync_copy(zbuf, acc.at[pl.ds(sid * SLICE, SLICE)])
        pltpu.sync_copy(dst_hbm.at[wid], dst_v)
        pltpu.sync_copy(ew_hbm.at[wid], ew_v)
        plsc.subcore_barrier()

        def body(j, c):
            pltpu.sync_copy(ew_v.at[j], acc.at[dst_v.at[j]], add=True)
            return c

        lax.fori_loop(0, CPT, body, 0)
        plsc.subcore_barrier()
        pltpu.sync_copy(acc.at[pl.ds(sid * SLICE, SLICE)],
                        out_hbm.at[cid, pl.ds(sid * SLICE, SLICE)])

    return deg_kernel(dst3, ew3)


def _agg_call(src3, dst3, ew3, h, zrs, D):
    @functools.partial(
        pl.kernel,
        out_type=jax.ShapeDtypeStruct((NC, NPAD, D), jnp.float32),
        mesh=_mesh(),
        compiler_params=pltpu.CompilerParams(use_tc_tiling_on_sc=False),
        scratch_types=[
            pltpu.VMEM((CPT, CHUNK), jnp.int32),
            pltpu.VMEM((CPT, CHUNK), jnp.int32),
            pltpu.VMEM((CPT, CHUNK), jnp.float32),
            pltpu.VMEM((2, CHUNK, D), jnp.float32),
            pltpu.VMEM((CHUNK, D), jnp.float32),
            pltpu.VMEM_SHARED((NPAD, D), jnp.float32),
            pltpu.SemaphoreType.DMA,
            pltpu.SemaphoreType.DMA,
        ],
    )
    def agg_kernel(src_hbm, dst_hbm, ew_hbm, h_hbm, z_hbm, out_hbm,
                   src_v, dst_v, ew_v, rows, obuf, acc, sem0, sem1):
        cid = lax.axis_index("c")
        sid = lax.axis_index("s")
        wid = sid * NC + cid
        sems = (sem0, sem1)
        nsl = pl.ds(sid * SLICE, SLICE)
        pltpu.sync_copy(z_hbm.at[nsl], acc.at[nsl])
        pltpu.sync_copy(src_hbm.at[wid], src_v)
        pltpu.sync_copy(dst_hbm.at[wid], dst_v)
        pltpu.sync_copy(ew_hbm.at[wid], ew_v)
        plsc.subcore_barrier()

        def fire(j, b):
            pltpu.async_copy(h_hbm.at[src_v.at[j]], rows.at[b], sems[b])

        def wait(j, b):
            pltpu.make_async_copy(h_hbm.at[src_v.at[j]], rows.at[b],
                                  sems[b]).wait()

        def scale(j, b):
            def kb(g, c):
                ewv = ew_v[j, pl.ds(g * L, L)]
                for e in range(L):
                    k = g * L + e
                    s = ewv[e]
                    for f in range(D // L):
                        sl = pl.ds(f * L, L)
                        rows[b, k, sl] = rows[b, k, sl] * s
                return c

            lax.fori_loop(0, CHUNK // L, kb, 0)

        def scat(j, b):
            pltpu.sync_copy(rows.at[b], acc.at[dst_v.at[j]], add=True)

        fire(0, 0)

        def outer(jo, c):
            j0 = 2 * jo
            fire(j0 + 1, 1)
            wait(j0, 0)
            scale(j0, 0)
            scat(j0, 0)

            @pl.when(j0 + 2 < CPT)
            def _():
                fire(j0 + 2, 0)

            wait(j0 + 1, 1)
            scale(j0 + 1, 1)
            scat(j0 + 1, 1)
            return c

        lax.fori_loop(0, CPT // 2, outer, 0)
        plsc.subcore_barrier()

        def oc(t, c):
            osl = pl.ds(sid * SLICE + t * CHUNK, CHUNK)
            pltpu.sync_copy(acc.at[osl], obuf)
            pltpu.sync_copy(obuf, out_hbm.at[cid, osl])
            return c

        lax.fori_loop(0, SLICE // CHUNK, oc, 0)

    return agg_kernel(src3, dst3, ew3, h, zrs)


def _tc_prep(degp, x_pad, W1):
    def body(deg_ref, x_ref, w_ref, dinv_ref, h_ref):
        deg = deg_ref[0, :] + deg_ref[1, :]
        dinv = jnp.where(deg > 0, lax.rsqrt(jnp.maximum(deg, 1e-12)), 0.0)
        dinv_ref[...] = dinv
        h = jnp.dot(x_ref[...], w_ref[...], preferred_element_type=jnp.float32)
        h_ref[...] = h * dinv[:, None]

    return pl.pallas_call(
        body,
        out_shape=(jax.ShapeDtypeStruct((NPAD,), jnp.float32),
                   jax.ShapeDtypeStruct((NPAD, HID), jnp.float32)),
    )(degp, x_pad, W1)


def _tc_mid(p1, dinv, b1, W2):
    def body(p_ref, dinv_ref, b_ref, w_ref, h2_ref):
        dinv = dinv_ref[...]
        s = (p_ref[0] + p_ref[1]) * dinv[:, None] + b_ref[...][None, :]
        a = jnp.maximum(s, 0.0)
        h2 = jnp.dot(a, w_ref[...], preferred_element_type=jnp.float32)
        h2_ref[...] = h2 * dinv[:, None]

    return pl.pallas_call(
        body,
        out_shape=jax.ShapeDtypeStruct((NPAD, OUT), jnp.float32),
    )(p1, dinv, b1, W2)


def _tc_fin(p2, dinv, b2):
    def body(p_ref, dinv_ref, b_ref, o_ref):
        o_ref[...] = ((p_ref[0] + p_ref[1]) * dinv_ref[...][:, None]
                      + b_ref[...][None, :])

    return pl.pallas_call(
        body,
        out_shape=jax.ShapeDtypeStruct((NPAD, OUT), jnp.float32),
    )(p2, dinv, b2)


def kernel(x, edge_index, edge_weight, W1, b1, W2, b2):
    loop = jnp.arange(N, dtype=jnp.int32)
    src = jnp.concatenate([edge_index[0].astype(jnp.int32), loop])
    dst = jnp.concatenate([edge_index[1].astype(jnp.int32), loop])
    ew = jnp.concatenate([edge_weight.astype(jnp.float32),
                          jnp.ones((N,), jnp.float32)])
    pad = E_PAD - (E_RAW + N)
    src3 = jnp.pad(src, (0, pad)).reshape(NW, CPT, CHUNK)
    dst3 = jnp.pad(dst, (0, pad)).reshape(NW, CPT, CHUNK)
    ew3 = jnp.pad(ew, (0, pad)).reshape(NW, CPT, CHUNK)
    x_pad = jnp.pad(x, ((0, NPAD - N), (0, 0)))

    degp = _deg_call(dst3, ew3)
    dinv, h1 = _tc_prep(degp, x_pad, W1)
    p1 = _agg_call(src3, dst3, ew3, h1,
                   jnp.zeros((NPAD, HID), jnp.float32), HID)
    h2 = _tc_mid(p1, dinv, b1, W2)
    p2 = _agg_call(src3, dst3, ew3, h2,
                   jnp.zeros((NPAD, OUT), jnp.float32), OUT)
    out = _tc_fin(p2, dinv, b2)
    return out[:N]
